# baseline (device time: 24651 ns/iter reference)
import jax
import jax.numpy as jnp
from jax import lax
from jax.experimental import pallas as pl
from jax.experimental.pallas import tpu as pltpu

E_LOCAL = 2
CHUNKS = 4


def kernel(x, assign, W1, W2):
    t, d = x.shape
    _, _, f = W1.shape

    my_x = lax.axis_index("x")
    eids = jnp.arange(E_LOCAL, dtype=jnp.int32)
    mask_self = (assign[:, None] == (E_LOCAL * my_x + eids)[None, :]).astype(
        jnp.float32
    )
    mask_nbr = (assign[:, None] == (E_LOCAL * (1 - my_x) + eids)[None, :]).astype(
        jnp.bfloat16
    )

    ch = t // CHUNKS

    def body(
        x_hbm,
        ms_ref,
        mn_ref,
        w1_hbm,
        w2_hbm,
        out_hbm,
        x_ref,
        w1_ref,
        w2_ref,
        outv_ref,
        xsd_ref,
        xr_ref,
        mr_ref,
        prc_ref,
        psd_ref,
        w_sems,
        out_sem,
        send_sems,
        recv_sems,
    ):
        mx = lax.axis_index("x")
        my = lax.axis_index("y")
        mz = lax.axis_index("z")
        nbr = (1 - mx, my, mz)

        x_dma = pltpu.make_async_copy(x_hbm, x_ref, w_sems.at[2])
        w1_dma = pltpu.make_async_copy(w1_hbm, w1_ref, w_sems.at[0])
        w2_dma = pltpu.make_async_copy(w2_hbm, w2_ref, w_sems.at[1])
        x_dma.start()
        w1_dma.start()
        w2_dma.start()

        x_dma.wait()
        xsd_ref[...] = x_ref[...].astype(jnp.bfloat16)

        barrier = pltpu.get_barrier_semaphore()
        pl.semaphore_signal(
            barrier, inc=1, device_id=nbr, device_id_type=pl.DeviceIdType.MESH
        )
        pl.semaphore_wait(barrier, 1)

        rdma_m = pltpu.make_async_remote_copy(
            src_ref=mn_ref,
            dst_ref=mr_ref,
            send_sem=send_sems.at[2 * CHUNKS],
            recv_sem=recv_sems.at[2 * CHUNKS],
            device_id=nbr,
            device_id_type=pl.DeviceIdType.MESH,
        )
        rdma_m.start()
        rdma_x = []
        for c in range(CHUNKS):
            r = pltpu.make_async_remote_copy(
                src_ref=xsd_ref.at[pl.ds(c * ch, ch)],
                dst_ref=xr_ref.at[pl.ds(c * ch, ch)],
                send_sem=send_sems.at[c],
                recv_sem=recv_sems.at[c],
                device_id=nbr,
                device_id_type=pl.DeviceIdType.MESH,
            )
            r.start()
            rdma_x.append(r)

        def ffn(xblk, mblk, rows):
            acc = jnp.zeros((rows, d), jnp.float32)
            for e in range(E_LOCAL):
                xe = xblk * mblk[:, e : e + 1]
                h = jnp.maximum(
                    jnp.dot(xe, w1_ref[e], preferred_element_type=jnp.float32),
                    0.0,
                )
                acc = acc + jnp.dot(
                    h, w2_ref[e], preferred_element_type=jnp.float32
                )
            return acc

        w1_dma.wait()
        w2_dma.wait()
        rdma_m.wait_recv()

        rdma_p = []
        for c in range(CHUNKS):
            rows = slice(c * ch, (c + 1) * ch)
            rdma_x[c].wait_recv()
            xblk = xr_ref[rows, :].astype(jnp.float32)
            mblk = mr_ref[rows, :].astype(jnp.float32)
            psd_ref[rows, :] = ffn(xblk, mblk, ch).astype(jnp.bfloat16)
            r = pltpu.make_async_remote_copy(
                src_ref=psd_ref.at[pl.ds(c * ch, ch)],
                dst_ref=prc_ref.at[pl.ds(c * ch, ch)],
                send_sem=send_sems.at[CHUNKS + c],
                recv_sem=recv_sems.at[CHUNKS + c],
                device_id=nbr,
                device_id_type=pl.DeviceIdType.MESH,
            )
            r.start()
            rdma_p.append(r)

        outv_ref[...] = ffn(x_ref[...], ms_ref[...], t)

        for c in range(CHUNKS):
            rows = slice(c * ch, (c + 1) * ch)
            rdma_p[c].wait_recv()
            outv_ref[rows, :] = outv_ref[rows, :] + prc_ref[rows, :].astype(
                jnp.float32
            )

        out_dma = pltpu.make_async_copy(outv_ref, out_hbm, out_sem)
        out_dma.start()

        rdma_m.wait_send()
        for r in rdma_x:
            r.wait_send()
        for r in rdma_p:
            r.wait_send()
        out_dma.wait()

    return pl.pallas_call(
        body,
        out_shape=jax.ShapeDtypeStruct((t, d), jnp.float32),
        in_specs=[
            pl.BlockSpec(memory_space=pltpu.MemorySpace.HBM),
            pl.BlockSpec(memory_space=pltpu.VMEM),
            pl.BlockSpec(memory_space=pltpu.VMEM),
            pl.BlockSpec(memory_space=pltpu.MemorySpace.HBM),
            pl.BlockSpec(memory_space=pltpu.MemorySpace.HBM),
        ],
        out_specs=pl.BlockSpec(memory_space=pltpu.MemorySpace.HBM),
        scratch_shapes=[
            pltpu.VMEM((t, d), jnp.float32),
            pltpu.VMEM((E_LOCAL, d, f), jnp.float32),
            pltpu.VMEM((E_LOCAL, f, d), jnp.float32),
            pltpu.VMEM((t, d), jnp.float32),
            pltpu.VMEM((t, d), jnp.bfloat16),
            pltpu.VMEM((t, d), jnp.bfloat16),
            pltpu.VMEM((t, E_LOCAL), jnp.bfloat16),
            pltpu.VMEM((t, d), jnp.bfloat16),
            pltpu.VMEM((t, d), jnp.bfloat16),
            pltpu.SemaphoreType.DMA((3,)),
            pltpu.SemaphoreType.DMA,
            pltpu.SemaphoreType.DMA((2 * CHUNKS + 1,)),
            pltpu.SemaphoreType.DMA((2 * CHUNKS + 1,)),
        ],
        compiler_params=pltpu.CompilerParams(collective_id=0),
    )(x, mask_self, mask_nbr, W1, W2)


# device time: 22853 ns/iter; 1.0787x vs baseline; 1.0787x over previous
import jax
import jax.numpy as jnp
from jax import lax
from jax.experimental import pallas as pl
from jax.experimental.pallas import tpu as pltpu

E_LOCAL = 2
CHUNKS = 4


def kernel(x, assign, W1, W2):
    t, d = x.shape
    _, _, f = W1.shape

    my_x = lax.axis_index("x")
    eids = jnp.arange(E_LOCAL, dtype=jnp.int32)
    mask_self = (assign[:, None] == (E_LOCAL * my_x + eids)[None, :]).astype(
        jnp.bfloat16
    )
    mask_nbr = (assign[:, None] == (E_LOCAL * (1 - my_x) + eids)[None, :]).astype(
        jnp.bfloat16
    )
    xb = x.astype(jnp.bfloat16)
    W1b = W1.astype(jnp.bfloat16)
    W2b = W2.astype(jnp.bfloat16)

    ch = t // CHUNKS

    def body(
        x_ref,
        ms_ref,
        mn_ref,
        w1_ref,
        w2_ref,
        out_hbm,
        outv_ref,
        xr_ref,
        mr_ref,
        prc_ref,
        psd_ref,
        out_sem,
        send_sems,
        recv_sems,
    ):
        mx = lax.axis_index("x")
        my = lax.axis_index("y")
        mz = lax.axis_index("z")
        nbr = (1 - mx, my, mz)

        barrier = pltpu.get_barrier_semaphore()
        pl.semaphore_signal(
            barrier, inc=1, device_id=nbr, device_id_type=pl.DeviceIdType.MESH
        )
        pl.semaphore_wait(barrier, 1)

        rdma_m = pltpu.make_async_remote_copy(
            src_ref=mn_ref,
            dst_ref=mr_ref,
            send_sem=send_sems.at[2 * CHUNKS],
            recv_sem=recv_sems.at[2 * CHUNKS],
            device_id=nbr,
            device_id_type=pl.DeviceIdType.MESH,
        )
        rdma_m.start()
        rdma_x = []
        for c in range(CHUNKS):
            r = pltpu.make_async_remote_copy(
                src_ref=x_ref.at[pl.ds(c * ch, ch)],
                dst_ref=xr_ref.at[pl.ds(c * ch, ch)],
                send_sem=send_sems.at[c],
                recv_sem=recv_sems.at[c],
                device_id=nbr,
                device_id_type=pl.DeviceIdType.MESH,
            )
            r.start()
            rdma_x.append(r)

        def ffn(xblk, mblk, rows):
            acc = jnp.zeros((rows, d), jnp.float32)
            for e in range(E_LOCAL):
                xe = xblk * mblk[:, e : e + 1]
                h = jnp.maximum(
                    jnp.dot(xe, w1_ref[e], preferred_element_type=jnp.float32),
                    0.0,
                ).astype(jnp.bfloat16)
                acc = acc + jnp.dot(
                    h, w2_ref[e], preferred_element_type=jnp.float32
                )
            return acc

        rdma_m.wait_recv()

        rdma_p = []
        for c in range(CHUNKS):
            rows = slice(c * ch, (c + 1) * ch)
            rdma_x[c].wait_recv()
            psd_ref[rows, :] = ffn(xr_ref[rows, :], mr_ref[rows, :], ch).astype(
                jnp.bfloat16
            )
            r = pltpu.make_async_remote_copy(
                src_ref=psd_ref.at[pl.ds(c * ch, ch)],
                dst_ref=prc_ref.at[pl.ds(c * ch, ch)],
                send_sem=send_sems.at[CHUNKS + c],
                recv_sem=recv_sems.at[CHUNKS + c],
                device_id=nbr,
                device_id_type=pl.DeviceIdType.MESH,
            )
            r.start()
            rdma_p.append(r)

        outv_ref[...] = ffn(x_ref[...], ms_ref[...], t)

        for c in range(CHUNKS):
            rows = slice(c * ch, (c + 1) * ch)
            rdma_p[c].wait_recv()
            outv_ref[rows, :] = outv_ref[rows, :] + prc_ref[rows, :].astype(
                jnp.float32
            )

        out_dma = pltpu.make_async_copy(outv_ref, out_hbm, out_sem)
        out_dma.start()

        rdma_m.wait_send()
        for r in rdma_x:
            r.wait_send()
        for r in rdma_p:
            r.wait_send()
        out_dma.wait()

    return pl.pallas_call(
        body,
        out_shape=jax.ShapeDtypeStruct((t, d), jnp.float32),
        in_specs=[pl.BlockSpec(memory_space=pltpu.VMEM)] * 5,
        out_specs=pl.BlockSpec(memory_space=pltpu.MemorySpace.HBM),
        scratch_shapes=[
            pltpu.VMEM((t, d), jnp.float32),
            pltpu.VMEM((t, d), jnp.bfloat16),
            pltpu.VMEM((t, E_LOCAL), jnp.bfloat16),
            pltpu.VMEM((t, d), jnp.bfloat16),
            pltpu.VMEM((t, d), jnp.bfloat16),
            pltpu.SemaphoreType.DMA,
            pltpu.SemaphoreType.DMA((2 * CHUNKS + 1,)),
            pltpu.SemaphoreType.DMA((2 * CHUNKS + 1,)),
        ],
        compiler_params=pltpu.CompilerParams(collective_id=0),
    )(xb, mask_self, mask_nbr, W1b, W2b)


# device time: 19621 ns/iter; 1.2564x vs baseline; 1.1647x over previous
import jax
import jax.numpy as jnp
from jax import lax
from jax.experimental import pallas as pl
from jax.experimental.pallas import tpu as pltpu

E_LOCAL = 2
CHUNKS = 4


def kernel(x, assign, W1, W2):
    t, d = x.shape
    _, _, f = W1.shape

    my_x = lax.axis_index("x")
    eids = jnp.arange(E_LOCAL, dtype=jnp.int32)
    mask_self = (assign[:, None] == (E_LOCAL * my_x + eids)[None, :]).astype(
        jnp.float32
    )
    mask_nbr = (assign[:, None] == (E_LOCAL * (1 - my_x) + eids)[None, :]).astype(
        jnp.bfloat16
    )

    hbm = pltpu.MemorySpace.HBM
    x = pltpu.with_memory_space_constraint(x, hbm)
    W1 = pltpu.with_memory_space_constraint(W1, hbm)
    W2 = pltpu.with_memory_space_constraint(W2, hbm)

    ch = t // CHUNKS

    def body(
        x_hbm,
        ms_ref,
        mn_ref,
        w1_hbm,
        w2_hbm,
        out_ref,
        x_ref,
        w1_ref,
        w2_ref,
        xsd_ref,
        xr_ref,
        mr_ref,
        prc_ref,
        psd_ref,
        in_sems,
        send_sems,
        recv_sems,
    ):
        mx = lax.axis_index("x")
        my = lax.axis_index("y")
        mz = lax.axis_index("z")
        nbr = (1 - mx, my, mz)

        x_dma = pltpu.make_async_copy(x_hbm, x_ref, in_sems.at[2])
        w1_dma = pltpu.make_async_copy(w1_hbm, w1_ref, in_sems.at[0])
        w2_dma = pltpu.make_async_copy(w2_hbm, w2_ref, in_sems.at[1])
        x_dma.start()
        w1_dma.start()
        w2_dma.start()

        x_dma.wait()
        xsd_ref[...] = x_ref[...].astype(jnp.bfloat16)

        barrier = pltpu.get_barrier_semaphore()
        pl.semaphore_signal(
            barrier, inc=1, device_id=nbr, device_id_type=pl.DeviceIdType.MESH
        )
        pl.semaphore_wait(barrier, 1)

        rdma_m = pltpu.make_async_remote_copy(
            src_ref=mn_ref,
            dst_ref=mr_ref,
            send_sem=send_sems.at[2 * CHUNKS],
            recv_sem=recv_sems.at[2 * CHUNKS],
            device_id=nbr,
            device_id_type=pl.DeviceIdType.MESH,
        )
        rdma_m.start()
        rdma_x = []
        for c in range(CHUNKS):
            r = pltpu.make_async_remote_copy(
                src_ref=xsd_ref.at[pl.ds(c * ch, ch)],
                dst_ref=xr_ref.at[pl.ds(c * ch, ch)],
                send_sem=send_sems.at[c],
                recv_sem=recv_sems.at[c],
                device_id=nbr,
                device_id_type=pl.DeviceIdType.MESH,
            )
            r.start()
            rdma_x.append(r)

        def ffn(xblk, mblk, rows):
            acc = jnp.zeros((rows, d), jnp.float32)
            for e in range(E_LOCAL):
                xe = xblk * mblk[:, e : e + 1]
                h = jnp.maximum(
                    jnp.dot(xe, w1_ref[e], preferred_element_type=jnp.float32),
                    0.0,
                )
                acc = acc + jnp.dot(
                    h, w2_ref[e], preferred_element_type=jnp.float32
                )
            return acc

        w1_dma.wait()
        w2_dma.wait()
        rdma_m.wait_recv()

        rdma_p = []
        for c in range(CHUNKS):
            rows = slice(c * ch, (c + 1) * ch)
            rdma_x[c].wait_recv()
            xblk = xr_ref[rows, :].astype(jnp.float32)
            mblk = mr_ref[rows, :].astype(jnp.float32)
            psd_ref[rows, :] = ffn(xblk, mblk, ch).astype(jnp.bfloat16)
            r = pltpu.make_async_remote_copy(
                src_ref=psd_ref.at[pl.ds(c * ch, ch)],
                dst_ref=prc_ref.at[pl.ds(c * ch, ch)],
                send_sem=send_sems.at[CHUNKS + c],
                recv_sem=recv_sems.at[CHUNKS + c],
                device_id=nbr,
                device_id_type=pl.DeviceIdType.MESH,
            )
            r.start()
            rdma_p.append(r)

        out_ref[...] = ffn(x_ref[...], ms_ref[...], t)

        for c in range(CHUNKS):
            rows = slice(c * ch, (c + 1) * ch)
            rdma_p[c].wait_recv()
            out_ref[rows, :] = out_ref[rows, :] + prc_ref[rows, :].astype(
                jnp.float32
            )

        rdma_m.wait_send()
        for r in rdma_x:
            r.wait_send()
        for r in rdma_p:
            r.wait_send()

    return pl.pallas_call(
        body,
        out_shape=jax.ShapeDtypeStruct((t, d), jnp.float32),
        in_specs=[
            pl.BlockSpec(memory_space=hbm),
            pl.BlockSpec(memory_space=pltpu.VMEM),
            pl.BlockSpec(memory_space=pltpu.VMEM),
            pl.BlockSpec(memory_space=hbm),
            pl.BlockSpec(memory_space=hbm),
        ],
        out_specs=pl.BlockSpec(memory_space=pltpu.VMEM),
        scratch_shapes=[
            pltpu.VMEM((t, d), jnp.float32),
            pltpu.VMEM((E_LOCAL, d, f), jnp.float32),
            pltpu.VMEM((E_LOCAL, f, d), jnp.float32),
            pltpu.VMEM((t, d), jnp.bfloat16),
            pltpu.VMEM((t, d), jnp.bfloat16),
            pltpu.VMEM((t, E_LOCAL), jnp.bfloat16),
            pltpu.VMEM((t, d), jnp.bfloat16),
            pltpu.VMEM((t, d), jnp.bfloat16),
            pltpu.SemaphoreType.DMA((3,)),
            pltpu.SemaphoreType.DMA((2 * CHUNKS + 1,)),
            pltpu.SemaphoreType.DMA((2 * CHUNKS + 1,)),
        ],
        compiler_params=pltpu.CompilerParams(collective_id=0),
    )(x, mask_self, mask_nbr, W1, W2)
